# baseline (speedup 1.0000x reference)
import functools

import jax
import jax.numpy as jnp
from jax import lax
from jax.experimental import pallas as pl
from jax.experimental.pallas import tpu as pltpu
from jax.experimental.pallas import tpu_sc as plsc

J = 0.1
_RT = 2400
_VS = 25600
_CR = 40

_NC, _NS = 2, 16
_NW = _NC * _NS
_L = 16


def _sc_body(xt_ref, lab_ref, cor_ref, sum_ref,
             lab_v, win_v, acc_v, gsem, buf0, buf1, s0, s1, sacc_v,
             *, B, V, b_per_w):
    wid = lax.axis_index("s") * _NC + lax.axis_index("c")
    base = wid * b_per_w

    pltpu.sync_copy(lab_ref.at[pl.ds(base, b_per_w)], lab_v)
    lchunks = [lab_v[pl.ds(c * _L, _L)] for c in range(b_per_w // _L)]
    rows = [lchunks[i // _L][i % _L] for i in range(b_per_w)]

    copies = []
    for i in range(b_per_w):
        panel = base + (i // _L) * _L
        copies.append(
            pltpu.async_copy(
                xt_ref.at[rows[i], pl.ds(panel, _L)], win_v.at[i], gsem)
        )
    for c in copies:
        c.wait()

    lane = lax.iota(jnp.int32, _L)
    acc = jnp.zeros((_L,), jnp.float32)
    for i in range(b_per_w):
        acc = acc + jnp.where(lane == i % _L, win_v[i, :], 0.0)
    acc_v[...] = acc
    pltpu.sync_copy(acc_v, cor_ref.at[pl.ds(wid * _L, _L)])

    nrow = _VS // _NW
    nch = nrow // _CR
    row0 = (V - _VS) + wid * nrow

    def _start(c, buf, sem):
        return pltpu.async_copy(
            xt_ref.at[pl.ds(row0 + c * _CR, _CR)], buf, sem)

    _start(0, buf0, s0)
    _start(1, buf1, s1)

    def _consume(buf, sem, a):
        pltpu.make_async_copy(xt_ref.at[pl.ds(0, _CR)], buf, sem).wait()

        def ib(i2, av):
            o = pl.multiple_of(i2 * _L, _L)
            vs = [buf[r, pl.ds(o, _L)] for r in range(_CR)]
            while len(vs) > 1:
                nxt = [vs[t] + vs[t + 1] for t in range(0, len(vs) - 1, 2)]
                if len(vs) % 2:
                    nxt.append(vs[-1])
                vs = nxt
            return av + vs[0]

        return lax.fori_loop(0, B // _L, ib, a)

    def ob(i, accs):
        a0, a1 = accs
        a0 = _consume(buf0, s0, a0)
        c2 = 2 * i + 2

        @pl.when(c2 < nch)
        def _():
            _start(c2, buf0, s0)

        a1 = _consume(buf1, s1, a1)
        c3 = 2 * i + 3

        @pl.when(c3 < nch)
        def _():
            _start(c3, buf1, s1)

        return (a0, a1)

    z = jnp.zeros((_L,), jnp.float32)
    a0, a1 = lax.fori_loop(0, nch // 2, ob, (z, z))
    sacc_v[...] = a0 + a1
    pltpu.sync_copy(sacc_v, sum_ref.at[pl.ds(wid * _L, _L)])


@functools.cache
def _make_sc(B, V):
    b_per_w = B // _NW
    mesh = plsc.VectorSubcoreMesh(
        core_axis_name="c", subcore_axis_name="s",
        num_cores=_NC, num_subcores=_NS,
    )
    return pl.kernel(
        functools.partial(_sc_body, B=B, V=V, b_per_w=b_per_w),
        out_type=(
            jax.ShapeDtypeStruct((_NW * _L,), jnp.float32),
            jax.ShapeDtypeStruct((_NW * _L,), jnp.float32),
        ),
        mesh=mesh,
        compiler_params=pltpu.CompilerParams(use_tc_tiling_on_sc=True),
        scratch_types=[
            pltpu.VMEM((b_per_w,), jnp.int32),
            pltpu.VMEM((b_per_w, _L), jnp.float32),
            pltpu.VMEM((_L,), jnp.float32),
            pltpu.SemaphoreType.DMA,
            pltpu.VMEM((_CR, B), jnp.float32),
            pltpu.VMEM((_CR, B), jnp.float32),
            pltpu.SemaphoreType.DMA,
            pltpu.SemaphoreType.DMA,
            pltpu.VMEM((_L,), jnp.float32),
        ],
    )


def _tc_body(x_ref, out_ref, acc_ref):
    k = pl.program_id(0)

    @pl.when(k == 0)
    def _init():
        acc_ref[0] = 0.0

    acc_ref[0] += jnp.sum(x_ref[...])

    @pl.when(k == pl.num_programs(0) - 1)
    def _fin():
        out_ref[...] = jnp.reshape(acc_ref[0], (1, 1))


def kernel(outputs, labels):
    B, V = outputs.shape
    lab = labels.astype(jnp.int32)
    xt = outputs.T

    cor_parts, sum_parts = _make_sc(B, V)(xt, lab)

    tc_sum = pl.pallas_call(
        _tc_body,
        grid=((V - _VS) // _RT,),
        in_specs=[pl.BlockSpec((_RT, B), lambda k: (k, 0))],
        out_specs=pl.BlockSpec((1, 1), lambda k: (0, 0)),
        out_shape=jax.ShapeDtypeStruct((1, 1), jnp.float32),
        scratch_shapes=[pltpu.SMEM((1,), jnp.float32)],
    )(xt)

    total = tc_sum[0, 0] + jnp.sum(sum_parts)
    csum = jnp.sum(cor_parts)
    return (total - (V + 1.0) * csum) / B + (2.0 * V - 1.0) * J


# speedup vs baseline: 1.0528x; 1.0528x over previous
import functools

import jax
import jax.numpy as jnp
from jax import lax
from jax.experimental import pallas as pl
from jax.experimental.pallas import tpu as pltpu
from jax.experimental.pallas import tpu_sc as plsc

J = 0.1
_RT = 5000

_NC, _NS = 2, 16
_NW = _NC * _NS
_L = 16


def _sc_body(xt_ref, lab_ref, cor_ref, lab_v, win_v, acc_v, gsem, *, b_per_w):
    wid = lax.axis_index("s") * _NC + lax.axis_index("c")
    base = wid * b_per_w

    pltpu.sync_copy(lab_ref.at[pl.ds(base, b_per_w)], lab_v)
    lchunks = [lab_v[pl.ds(c * _L, _L)] for c in range(b_per_w // _L)]
    rows = [lchunks[i // _L][i % _L] for i in range(b_per_w)]

    copies = []
    for i in range(b_per_w):
        panel = base + (i // _L) * _L
        copies.append(
            pltpu.async_copy(
                xt_ref.at[rows[i], pl.ds(panel, _L)], win_v.at[i], gsem)
        )
    for c in copies:
        c.wait()

    lane = lax.iota(jnp.int32, _L)
    acc = jnp.zeros((_L,), jnp.float32)
    for i in range(b_per_w):
        acc = acc + jnp.where(lane == i % _L, win_v[i, :], 0.0)
    acc_v[...] = acc
    pltpu.sync_copy(acc_v, cor_ref.at[pl.ds(wid * _L, _L)])


@functools.cache
def _make_sc(B, V):
    b_per_w = B // _NW
    mesh = plsc.VectorSubcoreMesh(
        core_axis_name="c", subcore_axis_name="s",
        num_cores=_NC, num_subcores=_NS,
    )
    return pl.kernel(
        functools.partial(_sc_body, b_per_w=b_per_w),
        out_type=jax.ShapeDtypeStruct((_NW * _L,), jnp.float32),
        mesh=mesh,
        compiler_params=pltpu.CompilerParams(use_tc_tiling_on_sc=True),
        scratch_types=[
            pltpu.VMEM((b_per_w,), jnp.int32),
            pltpu.VMEM((b_per_w, _L), jnp.float32),
            pltpu.VMEM((_L,), jnp.float32),
            pltpu.SemaphoreType.DMA,
        ],
    )


def _tc_body(x_ref, out_ref, acc_ref):
    k = pl.program_id(0)

    @pl.when(k == 0)
    def _init():
        acc_ref[0] = 0.0

    acc_ref[0] += jnp.sum(x_ref[...])

    @pl.when(k == pl.num_programs(0) - 1)
    def _fin():
        out_ref[...] = jnp.reshape(acc_ref[0], (1, 1))


def kernel(outputs, labels):
    B, V = outputs.shape
    lab = labels.astype(jnp.int32)
    xt = outputs.T

    cor_parts = _make_sc(B, V)(xt, lab)

    tc_sum = pl.pallas_call(
        _tc_body,
        grid=(V // _RT,),
        in_specs=[pl.BlockSpec((_RT, B), lambda k: (k, 0))],
        out_specs=pl.BlockSpec((1, 1), lambda k: (0, 0)),
        out_shape=jax.ShapeDtypeStruct((1, 1), jnp.float32),
        scratch_shapes=[pltpu.SMEM((1,), jnp.float32)],
    )(xt)

    csum = jnp.sum(cor_parts)
    return (tc_sum[0, 0] - (V + 1.0) * csum) / B + (2.0 * V - 1.0) * J
